# baseline (speedup 1.0000x reference)
import functools

import jax
import jax.numpy as jnp
from jax import lax
from jax.experimental import pallas as pl
from jax.experimental.pallas import tpu as pltpu
from jax.experimental.pallas import tpu_sc as plsc


def _build_sc_kernel(N, V, D, S, B):
    info = plsc.get_sparse_core_info()
    NC, NS, L = info.num_cores, info.num_subcores, info.num_lanes
    NW = NC * NS
    RPW = N // NW
    CH = 16
    NCH = RPW // CH
    NBK = B // CH
    NBUF = 4
    LOOK = 2
    assert N % NW == 0 and RPW % CH == 0 and B % CH == 0 and D % L == 0
    assert CH % L == 0 and CH % 8 == 0 and NCH % NBUF == 0

    mesh = plsc.VectorSubcoreMesh(core_axis_name="c", subcore_axis_name="s")

    @functools.partial(
        pl.kernel,
        mesh=mesh,
        out_type=jax.ShapeDtypeStruct((N, D), jnp.float32),
        scratch_types=[
            pltpu.VMEM((RPW,), jnp.int32),
            pltpu.VMEM((S, D), jnp.float32),
            pltpu.VMEM((NBUF, CH, D), jnp.float32),
            pltpu.VMEM((NBUF, 1, CH), jnp.int32),
            pltpu.SemaphoreType.DMA((NBUF,)),
            pltpu.SemaphoreType.DMA((NBUF,)),
        ],
    )
    def k(tid_hbm, tab_hbm, pos_hbm, out_hbm, tidv, posv, gbuf, nlv, gsem, osem):
        cid = lax.axis_index("c")
        sid = lax.axis_index("s")
        w = sid * NC + cid
        t0 = w * NCH
        iota = lax.iota(jnp.int32, L) * S

        def pos_row(j):
            s = (t0 + j) // NBK
            return [posv[s, pl.ds(c * L, L)] for c in range(D // L)]

        def start_gather_add(j, b):
            pv = pos_row(j)
            for r in range(CH):
                for c in range(D // L):
                    gbuf[b, r, pl.ds(c * L, L)] = pv[c]
            pltpu.async_copy(
                tab_hbm.at[tidv.at[pl.ds(j * CH, CH)]],
                gbuf.at[b],
                gsem.at[b],
                add=True,
            )

        def out_copy(b):
            return pltpu.make_async_copy(
                gbuf.at[b],
                out_hbm.at[nlv.at[b, 0]],
                osem.at[b],
            )

        pltpu.sync_copy(tid_hbm.at[pl.ds(t0 * CH, RPW)], tidv)
        pltpu.sync_copy(pos_hbm, posv)
        start_gather_add(0, 0)
        start_gather_add(1, 1)

        def step(j, b):
            b2 = (b + LOOK) % NBUF
            j2 = j + LOOK

            @pl.when(j2 < NCH)
            def _():
                @pl.when(j2 >= NBUF)
                def _():
                    out_copy(b2).wait()

                start_gather_add(j2, b2)

            t = t0 + j
            s = t // NBK
            off = (t % NBK) * CH * S + s

            pltpu.make_async_copy(
                tab_hbm.at[tidv.at[pl.ds(j * CH, CH)]], gbuf.at[b], gsem.at[b]
            ).wait()

            for c in range(CH // L):
                nlv[b, 0, pl.ds(c * L, L)] = iota + (off + c * L * S)

            out_copy(b).start()

        @pl.loop(0, NCH // NBUF)
        def _(grp):
            for b in range(NBUF):
                step(grp * NBUF + b, b)

        for b in range(NBUF):
            out_copy(b).wait()

    return k


def kernel(inputs, token_table, pos_table):
    B, S = inputs.shape
    V, D = token_table.shape
    N = B * S
    tid = inputs.T.reshape(N).astype(jnp.int32)
    run = _build_sc_kernel(N, V, D, S, B)
    out = run(tid, token_table, pos_table)
    return out.reshape(B, S, D)


# speedup vs baseline: 1.2757x; 1.2757x over previous
import functools

import jax
import jax.numpy as jnp
from jax import lax
from jax.experimental import pallas as pl
from jax.experimental.pallas import tpu as pltpu
from jax.experimental.pallas import tpu_sc as plsc


def _build_sc_kernel(N, V, D, S, B):
    info = plsc.get_sparse_core_info()
    NC, NS, L = info.num_cores, info.num_subcores, info.num_lanes
    NW = NC * NS
    RPW = N // NW
    CH = 32
    NCH = RPW // CH
    NBK = B // CH
    NBUF = 8
    LOOK = 4
    assert N % NW == 0 and RPW % CH == 0 and B % CH == 0 and D % L == 0
    assert CH % L == 0 and CH % 8 == 0 and NCH % NBUF == 0

    mesh = plsc.VectorSubcoreMesh(core_axis_name="c", subcore_axis_name="s")

    @functools.partial(
        pl.kernel,
        mesh=mesh,
        out_type=jax.ShapeDtypeStruct((N, D), jnp.float32),
        scratch_types=[
            pltpu.VMEM((RPW,), jnp.int32),
            pltpu.VMEM((S, D), jnp.float32),
            pltpu.VMEM((NBUF, CH, D), jnp.float32),
            pltpu.VMEM((NBUF, 1, CH), jnp.int32),
            pltpu.SemaphoreType.DMA((NBUF,)),
            pltpu.SemaphoreType.DMA((NBUF,)),
        ],
    )
    def k(tid_hbm, tab_hbm, pos_hbm, out_hbm, tidv, posv, gbuf, nlv, gsem, osem):
        cid = lax.axis_index("c")
        sid = lax.axis_index("s")
        w = sid * NC + cid
        t0 = w * NCH
        iota = lax.iota(jnp.int32, L) * S

        def pos_row(j):
            s = (t0 + j) // NBK
            return [posv[s, pl.ds(c * L, L)] for c in range(D // L)]

        def start_gather_add(j, b):
            pv = pos_row(j)
            for r in range(CH):
                for c in range(D // L):
                    gbuf[b, r, pl.ds(c * L, L)] = pv[c]
            pltpu.async_copy(
                tab_hbm.at[tidv.at[pl.ds(j * CH, CH)]],
                gbuf.at[b],
                gsem.at[b],
                add=True,
            )

        def out_copy(b):
            return pltpu.make_async_copy(
                gbuf.at[b],
                out_hbm.at[nlv.at[b, 0]],
                osem.at[b],
            )

        pltpu.sync_copy(tid_hbm.at[pl.ds(t0 * CH, RPW)], tidv)
        pltpu.sync_copy(pos_hbm, posv)
        for j0 in range(LOOK):
            start_gather_add(j0, j0)

        def step(j, b):
            b2 = (b + LOOK) % NBUF
            j2 = j + LOOK

            @pl.when(j2 < NCH)
            def _():
                @pl.when(j2 >= NBUF)
                def _():
                    out_copy(b2).wait()

                start_gather_add(j2, b2)

            t = t0 + j
            s = t // NBK
            off = (t % NBK) * CH * S + s

            pltpu.make_async_copy(
                tab_hbm.at[tidv.at[pl.ds(j * CH, CH)]], gbuf.at[b], gsem.at[b]
            ).wait()

            for c in range(CH // L):
                nlv[b, 0, pl.ds(c * L, L)] = iota + (off + c * L * S)

            out_copy(b).start()

        @pl.loop(0, NCH // NBUF)
        def _(grp):
            for b in range(NBUF):
                step(grp * NBUF + b, b)

        for b in range(NBUF):
            out_copy(b).wait()

    return k


def kernel(inputs, token_table, pos_table):
    B, S = inputs.shape
    V, D = token_table.shape
    N = B * S
    tid = inputs.T.reshape(N).astype(jnp.int32)
    run = _build_sc_kernel(N, V, D, S, B)
    out = run(tid, token_table, pos_table)
    return out.reshape(B, S, D)


# speedup vs baseline: 1.3455x; 1.0547x over previous
import functools

import jax
import jax.numpy as jnp
from jax import lax
from jax.experimental import pallas as pl
from jax.experimental.pallas import tpu as pltpu
from jax.experimental.pallas import tpu_sc as plsc


def _build_sc_kernel(N, V, D, S, B):
    info = plsc.get_sparse_core_info()
    NC, NS, L = info.num_cores, info.num_subcores, info.num_lanes
    NW = NC * NS
    RPW = N // NW
    CH = 32
    NCH = RPW // CH
    NBK = B // CH
    NBUF = 4
    LOOK = 2
    assert N % NW == 0 and RPW % CH == 0 and B % CH == 0 and D % L == 0
    assert CH % L == 0 and CH % 8 == 0 and NCH % NBUF == 0

    mesh = plsc.VectorSubcoreMesh(core_axis_name="c", subcore_axis_name="s")

    @functools.partial(
        pl.kernel,
        mesh=mesh,
        out_type=jax.ShapeDtypeStruct((N, D), jnp.float32),
        scratch_types=[
            pltpu.VMEM((RPW,), jnp.int32),
            pltpu.VMEM((S, D), jnp.float32),
            pltpu.VMEM((NBUF, CH, D), jnp.float32),
            pltpu.VMEM((NBUF, 1, CH), jnp.int32),
            pltpu.SemaphoreType.DMA((NBUF,)),
            pltpu.SemaphoreType.DMA((NBUF,)),
        ],
    )
    def k(tid_hbm, tab_hbm, pos_hbm, out_hbm, tidv, posv, gbuf, nlv, gsem, osem):
        cid = lax.axis_index("c")
        sid = lax.axis_index("s")
        w = sid * NC + cid
        t0 = w * NCH
        iota = lax.iota(jnp.int32, L) * S

        def pos_row(j):
            s = (t0 + j) // NBK
            return [posv[s, pl.ds(c * L, L)] for c in range(D // L)]

        def start_gather_add(j, b):
            pv = pos_row(j)
            for r in range(CH):
                for c in range(D // L):
                    gbuf[b, r, pl.ds(c * L, L)] = pv[c]
            pltpu.async_copy(
                tab_hbm.at[tidv.at[pl.ds(j * CH, CH)]],
                gbuf.at[b],
                gsem.at[b],
                add=True,
            )

        def out_copy(b):
            return pltpu.make_async_copy(
                gbuf.at[b],
                out_hbm.at[nlv.at[b, 0]],
                osem.at[b],
            )

        pltpu.sync_copy(tid_hbm.at[pl.ds(t0 * CH, RPW)], tidv)
        pltpu.sync_copy(pos_hbm, posv)
        for j0 in range(LOOK):
            start_gather_add(j0, j0)

        def step(j, b):
            b2 = (b + LOOK) % NBUF
            j2 = j + LOOK

            @pl.when(j2 < NCH)
            def _():
                @pl.when(j2 >= NBUF)
                def _():
                    out_copy(b2).wait()

                start_gather_add(j2, b2)

            t = t0 + j
            s = t // NBK
            off = (t % NBK) * CH * S + s

            pltpu.make_async_copy(
                tab_hbm.at[tidv.at[pl.ds(j * CH, CH)]], gbuf.at[b], gsem.at[b]
            ).wait()

            for c in range(CH // L):
                nlv[b, 0, pl.ds(c * L, L)] = iota + (off + c * L * S)

            out_copy(b).start()

        @pl.loop(0, NCH // NBUF)
        def _(grp):
            for b in range(NBUF):
                step(grp * NBUF + b, b)

        for b in range(NBUF):
            out_copy(b).wait()

    return k


def kernel(inputs, token_table, pos_table):
    B, S = inputs.shape
    V, D = token_table.shape
    N = B * S
    tid = inputs.T.reshape(N).astype(jnp.int32)
    run = _build_sc_kernel(N, V, D, S, B)
    out = run(tid, token_table, pos_table)
    return out.reshape(B, S, D)
